# Initial kernel scaffold; baseline (speedup 1.0000x reference)
import jax
import jax.numpy as jnp
from jax.experimental import pallas as pl


def kernel(x, edge_index, W1, b1, Wmu, bmu, Wsig, bsig, gnoise):
    raise NotImplementedError("write your pallas kernel here")


# speedup vs baseline: 5.6219x; 5.6219x over previous
import functools

import jax
import jax.numpy as jnp
from jax import lax
from jax.experimental import pallas as pl
from jax.experimental.pallas import tpu as pltpu
from jax.experimental.pallas import tpu_sc as plsc

N = 10000
NPAD = 10240
IN_DIM = 256
FEAT = 256
LAT = 64
E = 160000
EPAD = 163840
PAD_IDX = NPAD - 1

NC, NS, L = 2, 16, 16
CHUNK = 128
CHUNKS = EPAD // NS // CHUNK
RPT = NPAD // NS
EPW = EPAD // (NC * NS)


def _mesh():
    return plsc.VectorSubcoreMesh(
        core_axis_name="c", subcore_axis_name="s",
        num_cores=NC, num_subcores=NS)


def _deg_body(dst_hbm, hist_hbm, dst_v, hist_v):
    c = lax.axis_index("c")
    s = lax.axis_index("s")
    w = s * NC + c
    pltpu.sync_copy(dst_hbm.at[w], dst_v)
    zeros = jnp.zeros((L,), jnp.float32)
    ones = jnp.ones((L,), jnp.float32)

    @pl.loop(0, NPAD // L)
    def _zero(i):
        hist_v[pl.ds(i * L, L)] = zeros

    @pl.loop(0, EPW // L)
    def _count(i):
        idx = dst_v[pl.ds(i * L, L)]
        plsc.addupdate_scatter(hist_v, [idx], ones)

    pltpu.sync_copy(hist_v, hist_hbm.at[w])


@functools.cache
def _deg_kernel():
    return functools.partial(
        pl.kernel,
        out_type=jax.ShapeDtypeStruct((NC * NS, NPAD), jnp.float32),
        mesh=_mesh(),
        scratch_types=[
            pltpu.VMEM((EPW,), jnp.int32),
            pltpu.VMEM((NPAD,), jnp.float32),
        ],
        compiler_params=pltpu.CompilerParams(needs_layout_passes=False),
    )(_deg_body)


def _make_scatter(edge_split):
    DH = 128
    chunks = CHUNKS // 2 if edge_split else CHUNKS

    def body(table_hbm, src_hbm, dst_hbm, out_hbm, src_v, dst_v, rows_v, sem, acc):
        c = lax.axis_index("c")
        s = lax.axis_index("s")
        w = c * NS + s if edge_split else s
        pltpu.sync_copy(src_hbm.at[w], src_v)
        pltpu.sync_copy(dst_hbm.at[w], dst_v)

        zeros = jnp.zeros((L,), jnp.float32)

        @pl.loop(0, CHUNK)
        def _zr(r):
            @pl.loop(0, DH // L)
            def _zc(k):
                rows_v[r, pl.ds(k * L, L)] = zeros

        for k in range(RPT // CHUNK):
            pltpu.sync_copy(rows_v, acc.at[pl.ds(s * RPT + k * CHUNK, CHUNK)])
        plsc.subcore_barrier()

        table = table_hbm if edge_split else table_hbm.at[c]

        @pl.loop(0, chunks)
        def _edge(j):
            pltpu.async_copy(table.at[src_v.at[j]], rows_v, sem).wait()
            pltpu.sync_copy(rows_v, acc.at[dst_v.at[j]], add=True)

        plsc.subcore_barrier()
        pltpu.sync_copy(acc.at[pl.ds(s * RPT, RPT)],
                        out_hbm.at[c, pl.ds(s * RPT, RPT)])

    tshape = (NPAD, DH) if edge_split else (NC, NPAD, DH)
    nw = NC * NS if edge_split else NS
    return functools.partial(
        pl.kernel,
        out_type=jax.ShapeDtypeStruct((NC, NPAD, DH), jnp.float32),
        mesh=_mesh(),
        scratch_types=[
            pltpu.VMEM((chunks, CHUNK), jnp.int32),
            pltpu.VMEM((chunks, CHUNK), jnp.int32),
            pltpu.VMEM((CHUNK, DH), jnp.float32),
            pltpu.SemaphoreType.DMA,
            pltpu.VMEM_SHARED((NPAD, DH), jnp.float32),
        ],
        compiler_params=pltpu.CompilerParams(needs_layout_passes=False),
    )(body)


_scatter_kernel = functools.cache(_make_scatter)


_BM = 512


def _mm1_body(x_ref, w_ref, o_ref):
    o_ref[...] = jnp.dot(x_ref[...], w_ref[...],
                         preferred_element_type=jnp.float32)


def _mm1(x, w):
    return pl.pallas_call(
        _mm1_body,
        grid=(NPAD // _BM,),
        in_specs=[pl.BlockSpec((_BM, IN_DIM), lambda i: (i, 0)),
                  pl.BlockSpec((IN_DIM, FEAT), lambda i: (0, 0))],
        out_specs=pl.BlockSpec((_BM, FEAT), lambda i: (i, 0)),
        out_shape=jax.ShapeDtypeStruct((NPAD, FEAT), jnp.float32),
    )(x, w)


def _scale_body(hist_ref, xw_ref, dinv_ref, ycat_ref):
    deg = jnp.sum(hist_ref[...], axis=0) + 1.0
    dinv = lax.rsqrt(jnp.maximum(deg, 1.0))
    y = xw_ref[...] * dinv[:, None]
    dinv_ref[...] = dinv
    ycat_ref[0] = y[:, : FEAT // 2]
    ycat_ref[1] = y[:, FEAT // 2:]


def _scale(hist, xw):
    return pl.pallas_call(
        _scale_body,
        grid=(NPAD // _BM,),
        in_specs=[pl.BlockSpec((NC * NS, _BM), lambda i: (0, i)),
                  pl.BlockSpec((_BM, FEAT), lambda i: (i, 0))],
        out_specs=[pl.BlockSpec((_BM,), lambda i: (i,)),
                   pl.BlockSpec((NC, _BM, FEAT // 2), lambda i: (0, i, 0))],
        out_shape=[jax.ShapeDtypeStruct((NPAD,), jnp.float32),
                   jax.ShapeDtypeStruct((NC, NPAD, FEAT // 2), jnp.float32)],
    )(hist, xw)


def _layer2_body(s1_ref, y1_ref, dinv_ref, b1_ref, wcat_ref, y2_ref, i_ref=None):
    i = pl.program_id(0)
    s = jnp.concatenate([s1_ref[0], s1_ref[1]], axis=1)
    y1 = jnp.concatenate([y1_ref[0], y1_ref[1]], axis=1)
    dinv = dinv_ref[...]
    h = jnp.maximum(dinv[:, None] * (s + y1) + b1_ref[...][None, :], 0.0)
    row = i * _BM + lax.broadcasted_iota(jnp.int32, (_BM, 1), 0)
    h = jnp.where(row < N, h, 0.0)
    c = jnp.dot(h, wcat_ref[...], preferred_element_type=jnp.float32)
    y2_ref[...] = c * dinv[:, None]


def _layer2(s1cat, y1cat, dinv, b1, wcat):
    return pl.pallas_call(
        _layer2_body,
        grid=(NPAD // _BM,),
        in_specs=[pl.BlockSpec((NC, _BM, FEAT // 2), lambda i: (0, i, 0)),
                  pl.BlockSpec((NC, _BM, FEAT // 2), lambda i: (0, i, 0)),
                  pl.BlockSpec((_BM,), lambda i: (i,)),
                  pl.BlockSpec((FEAT,), lambda i: (0,)),
                  pl.BlockSpec((FEAT, 2 * LAT), lambda i: (0, 0))],
        out_specs=pl.BlockSpec((_BM, 2 * LAT), lambda i: (i, 0)),
        out_shape=jax.ShapeDtypeStruct((NPAD, 2 * LAT), jnp.float32),
    )(s1cat, y1cat, dinv, b1, wcat)


def _zcomp_body(s2_ref, y2_ref, dinv_ref, bcat_ref, gn_ref, z_ref):
    s = s2_ref[0] + s2_ref[1]
    y2 = y2_ref[...]
    o = dinv_ref[...][:, None] * (s + y2) + bcat_ref[...][None, :]
    xu = o[:, :LAT]
    xs = o[:, LAT:]
    z_ref[...] = gn_ref[...] * jnp.exp(xs) + xu


def _zcomp(s2cat, y2cat, dinv, bcat, gn):
    return pl.pallas_call(
        _zcomp_body,
        grid=(NPAD // _BM,),
        in_specs=[pl.BlockSpec((NC, _BM, 2 * LAT), lambda i: (0, i, 0)),
                  pl.BlockSpec((_BM, 2 * LAT), lambda i: (i, 0)),
                  pl.BlockSpec((_BM,), lambda i: (i,)),
                  pl.BlockSpec((2 * LAT,), lambda i: (0,)),
                  pl.BlockSpec((_BM, LAT), lambda i: (i, 0))],
        out_specs=pl.BlockSpec((_BM, LAT), lambda i: (i, 0)),
        out_shape=jax.ShapeDtypeStruct((NPAD, LAT), jnp.float32),
    )(s2cat, y2cat, dinv, bcat, gn)


_BD = 512


def _decode_body(zr_ref, zc_ref, o_ref):
    p = lax.dot_general(zr_ref[...], zc_ref[...],
                        (((1,), (1,)), ((), ())),
                        preferred_element_type=jnp.float32)
    o_ref[...] = jax.nn.sigmoid(p)


def _decode(z):
    g = NPAD // _BD
    return pl.pallas_call(
        _decode_body,
        grid=(g, g),
        in_specs=[pl.BlockSpec((_BD, LAT), lambda i, j: (i, 0)),
                  pl.BlockSpec((_BD, LAT), lambda i, j: (j, 0))],
        out_specs=pl.BlockSpec((_BD, _BD), lambda i, j: (i, j)),
        out_shape=jax.ShapeDtypeStruct((N, N), jnp.float32),
    )(z, z)


def kernel(x, edge_index, W1, b1, Wmu, bmu, Wsig, bsig, gnoise):
    ei = edge_index.astype(jnp.int32)
    pad = jnp.full((EPAD - E,), PAD_IDX, jnp.int32)
    src = jnp.concatenate([ei[0], pad])
    dst = jnp.concatenate([ei[1], pad])
    src_sc = src.reshape(NS, CHUNKS, CHUNK)
    dst_sc = dst.reshape(NS, CHUNKS, CHUNK)
    src_es = src.reshape(NC * NS, CHUNKS // 2, CHUNK)
    dst_es = dst.reshape(NC * NS, CHUNKS // 2, CHUNK)
    dst_deg = dst.reshape(NC * NS, EPW)

    x_pad = jnp.pad(x, ((0, NPAD - N), (0, 0)))
    gn_pad = jnp.pad(gnoise, ((0, NPAD - N), (0, 0)))
    wcat = jnp.concatenate([Wmu, Wsig], axis=1)
    bcat = jnp.concatenate([bmu, bsig])

    hist = _deg_kernel()(dst_deg)
    xw = _mm1(x_pad, W1)
    dinv, y1cat = _scale(hist, xw)
    s1cat = _scatter_kernel(False)(y1cat, src_sc, dst_sc)
    y2 = _layer2(s1cat, y1cat, dinv, b1, wcat)
    s2cat = _scatter_kernel(True)(y2, src_es, dst_es)
    z = _zcomp(s2cat, y2, dinv, bcat, gn_pad)
    return _decode(z)


# speedup vs baseline: 6.0083x; 1.0687x over previous
import functools

import jax
import jax.numpy as jnp
from jax import lax
from jax.experimental import pallas as pl
from jax.experimental.pallas import tpu as pltpu
from jax.experimental.pallas import tpu_sc as plsc

N = 10000
NPAD = 10240
IN_DIM = 256
FEAT = 256
LAT = 64
E = 160000
EPAD = 163840
PAD_IDX = NPAD - 1

NC, NS, L = 2, 16, 16
CHUNK = 128
CHUNKS = EPAD // NS // CHUNK
RPT = NPAD // NS
EPW = EPAD // (NC * NS)


def _mesh():
    return plsc.VectorSubcoreMesh(
        core_axis_name="c", subcore_axis_name="s",
        num_cores=NC, num_subcores=NS)


def _deg_body(dst_hbm, hist_hbm, dst_v, hist_v):
    c = lax.axis_index("c")
    s = lax.axis_index("s")
    w = s * NC + c
    pltpu.sync_copy(dst_hbm.at[w], dst_v)
    zeros = jnp.zeros((L,), jnp.float32)
    ones = jnp.ones((L,), jnp.float32)

    @pl.loop(0, NPAD // L)
    def _zero(i):
        hist_v[pl.ds(i * L, L)] = zeros

    @pl.loop(0, EPW // L)
    def _count(i):
        idx = dst_v[pl.ds(i * L, L)]
        plsc.addupdate_scatter(hist_v, [idx], ones)

    pltpu.sync_copy(hist_v, hist_hbm.at[w])


@functools.cache
def _deg_kernel():
    return functools.partial(
        pl.kernel,
        out_type=jax.ShapeDtypeStruct((NC * NS, NPAD), jnp.float32),
        mesh=_mesh(),
        scratch_types=[
            pltpu.VMEM((EPW,), jnp.int32),
            pltpu.VMEM((NPAD,), jnp.float32),
        ],
        compiler_params=pltpu.CompilerParams(needs_layout_passes=False),
    )(_deg_body)


def _make_scatter(edge_split):
    DH = 128
    chunks = CHUNKS // 2 if edge_split else CHUNKS

    def body(table_hbm, src_hbm, dst_hbm, out_hbm,
             src_a, src_b, dst_v, rows_a, rows_b,
             sem_ga, sem_gb, sem_ia, sem_ib, acc):
        c = lax.axis_index("c")
        s = lax.axis_index("s")
        w = c * NS + s if edge_split else s
        pltpu.sync_copy(dst_hbm.at[w], dst_v)

        zeros = jnp.zeros((L,), jnp.float32)

        @pl.loop(0, CHUNK)
        def _zr(r):
            @pl.loop(0, DH // L)
            def _zc(k):
                rows_a[r, pl.ds(k * L, L)] = zeros

        for k in range(RPT // CHUNK):
            pltpu.sync_copy(rows_a, acc.at[pl.ds(s * RPT + k * CHUNK, CHUNK)])

        table = table_hbm if edge_split else table_hbm.at[c]
        last = chunks - 1

        def gather(sbuf, rbuf, sem):
            pltpu.async_copy(table.at[sbuf], rbuf, sem)

        def gwait(rbuf, sem):
            pltpu.make_async_copy(table.at[src_a], rbuf, sem).wait()

        def iload(jj, sbuf, sem):
            pltpu.async_copy(src_hbm.at[w, jj], sbuf, sem)

        def iwait(sbuf, sem):
            pltpu.make_async_copy(src_hbm.at[w, 0], sbuf, sem).wait()

        pltpu.sync_copy(src_hbm.at[w, 0], src_a)
        gather(src_a, rows_a, sem_ga)
        iload(1, src_b, sem_ib)
        iwait(src_b, sem_ib)
        plsc.subcore_barrier()

        @pl.loop(0, chunks, step=2)
        def _edge(j):
            gwait(rows_a, sem_ga)
            gather(src_b, rows_b, sem_gb)
            iload(jnp.minimum(j + 2, last), src_a, sem_ia)
            pltpu.sync_copy(rows_a, acc.at[dst_v.at[j]], add=True)
            iwait(src_a, sem_ia)
            gwait(rows_b, sem_gb)
            gather(src_a, rows_a, sem_ga)
            iload(jnp.minimum(j + 3, last), src_b, sem_ib)
            pltpu.sync_copy(rows_b, acc.at[dst_v.at[j + 1]], add=True)
            iwait(src_b, sem_ib)

        gwait(rows_a, sem_ga)
        plsc.subcore_barrier()
        pltpu.sync_copy(acc.at[pl.ds(s * RPT, RPT)],
                        out_hbm.at[c, pl.ds(s * RPT, RPT)])

    tshape = (NPAD, DH) if edge_split else (NC, NPAD, DH)
    nw = NC * NS if edge_split else NS
    return functools.partial(
        pl.kernel,
        out_type=jax.ShapeDtypeStruct((NC, NPAD, DH), jnp.float32),
        mesh=_mesh(),
        scratch_types=[
            pltpu.VMEM((CHUNK,), jnp.int32),
            pltpu.VMEM((CHUNK,), jnp.int32),
            pltpu.VMEM((chunks, CHUNK), jnp.int32),
            pltpu.VMEM((CHUNK, DH), jnp.float32),
            pltpu.VMEM((CHUNK, DH), jnp.float32),
            pltpu.SemaphoreType.DMA,
            pltpu.SemaphoreType.DMA,
            pltpu.SemaphoreType.DMA,
            pltpu.SemaphoreType.DMA,
            pltpu.VMEM_SHARED((NPAD, DH), jnp.float32),
        ],
        compiler_params=pltpu.CompilerParams(needs_layout_passes=False),
    )(body)


_scatter_kernel = functools.cache(_make_scatter)


_BM = 512


def _mm1_body(x_ref, w_ref, o_ref):
    o_ref[...] = jnp.dot(x_ref[...], w_ref[...],
                         preferred_element_type=jnp.float32)


def _mm1(x, w):
    return pl.pallas_call(
        _mm1_body,
        grid=(NPAD // _BM,),
        in_specs=[pl.BlockSpec((_BM, IN_DIM), lambda i: (i, 0)),
                  pl.BlockSpec((IN_DIM, FEAT), lambda i: (0, 0))],
        out_specs=pl.BlockSpec((_BM, FEAT), lambda i: (i, 0)),
        out_shape=jax.ShapeDtypeStruct((NPAD, FEAT), jnp.float32),
    )(x, w)


def _scale_body(hist_ref, xw_ref, dinv_ref, ycat_ref):
    deg = jnp.sum(hist_ref[...], axis=0) + 1.0
    dinv = lax.rsqrt(jnp.maximum(deg, 1.0))
    y = xw_ref[...] * dinv[:, None]
    dinv_ref[...] = dinv
    ycat_ref[0] = y[:, : FEAT // 2]
    ycat_ref[1] = y[:, FEAT // 2:]


def _scale(hist, xw):
    return pl.pallas_call(
        _scale_body,
        grid=(NPAD // _BM,),
        in_specs=[pl.BlockSpec((NC * NS, _BM), lambda i: (0, i)),
                  pl.BlockSpec((_BM, FEAT), lambda i: (i, 0))],
        out_specs=[pl.BlockSpec((_BM,), lambda i: (i,)),
                   pl.BlockSpec((NC, _BM, FEAT // 2), lambda i: (0, i, 0))],
        out_shape=[jax.ShapeDtypeStruct((NPAD,), jnp.float32),
                   jax.ShapeDtypeStruct((NC, NPAD, FEAT // 2), jnp.float32)],
    )(hist, xw)


def _layer2_body(s1_ref, y1_ref, dinv_ref, b1_ref, wcat_ref, y2_ref, i_ref=None):
    i = pl.program_id(0)
    s = jnp.concatenate([s1_ref[0], s1_ref[1]], axis=1)
    y1 = jnp.concatenate([y1_ref[0], y1_ref[1]], axis=1)
    dinv = dinv_ref[...]
    h = jnp.maximum(dinv[:, None] * (s + y1) + b1_ref[...][None, :], 0.0)
    row = i * _BM + lax.broadcasted_iota(jnp.int32, (_BM, 1), 0)
    h = jnp.where(row < N, h, 0.0)
    c = jnp.dot(h, wcat_ref[...], preferred_element_type=jnp.float32)
    y2_ref[...] = c * dinv[:, None]


def _layer2(s1cat, y1cat, dinv, b1, wcat):
    return pl.pallas_call(
        _layer2_body,
        grid=(NPAD // _BM,),
        in_specs=[pl.BlockSpec((NC, _BM, FEAT // 2), lambda i: (0, i, 0)),
                  pl.BlockSpec((NC, _BM, FEAT // 2), lambda i: (0, i, 0)),
                  pl.BlockSpec((_BM,), lambda i: (i,)),
                  pl.BlockSpec((FEAT,), lambda i: (0,)),
                  pl.BlockSpec((FEAT, 2 * LAT), lambda i: (0, 0))],
        out_specs=pl.BlockSpec((_BM, 2 * LAT), lambda i: (i, 0)),
        out_shape=jax.ShapeDtypeStruct((NPAD, 2 * LAT), jnp.float32),
    )(s1cat, y1cat, dinv, b1, wcat)


def _zcomp_body(s2_ref, y2_ref, dinv_ref, bcat_ref, gn_ref, z_ref):
    s = s2_ref[0] + s2_ref[1]
    y2 = y2_ref[...]
    o = dinv_ref[...][:, None] * (s + y2) + bcat_ref[...][None, :]
    xu = o[:, :LAT]
    xs = o[:, LAT:]
    z_ref[...] = gn_ref[...] * jnp.exp(xs) + xu


def _zcomp(s2cat, y2cat, dinv, bcat, gn):
    return pl.pallas_call(
        _zcomp_body,
        grid=(NPAD // _BM,),
        in_specs=[pl.BlockSpec((NC, _BM, 2 * LAT), lambda i: (0, i, 0)),
                  pl.BlockSpec((_BM, 2 * LAT), lambda i: (i, 0)),
                  pl.BlockSpec((_BM,), lambda i: (i,)),
                  pl.BlockSpec((2 * LAT,), lambda i: (0,)),
                  pl.BlockSpec((_BM, LAT), lambda i: (i, 0))],
        out_specs=pl.BlockSpec((_BM, LAT), lambda i: (i, 0)),
        out_shape=jax.ShapeDtypeStruct((NPAD, LAT), jnp.float32),
    )(s2cat, y2cat, dinv, bcat, gn)


_BD = 512


def _decode_body(zr_ref, zc_ref, o_ref):
    p = lax.dot_general(zr_ref[...], zc_ref[...],
                        (((1,), (1,)), ((), ())),
                        preferred_element_type=jnp.float32)
    o_ref[...] = 0.5 * jnp.tanh(0.5 * p) + 0.5


def _decode(z):
    g = NPAD // _BD
    return pl.pallas_call(
        _decode_body,
        grid=(g, g),
        in_specs=[pl.BlockSpec((_BD, LAT), lambda i, j: (i, 0)),
                  pl.BlockSpec((_BD, LAT), lambda i, j: (j, 0))],
        out_specs=pl.BlockSpec((_BD, _BD), lambda i, j: (i, j)),
        out_shape=jax.ShapeDtypeStruct((N, N), jnp.float32),
    )(z, z)


def kernel(x, edge_index, W1, b1, Wmu, bmu, Wsig, bsig, gnoise):
    ei = edge_index.astype(jnp.int32)
    pad = jnp.full((EPAD - E,), PAD_IDX, jnp.int32)
    src = jnp.concatenate([ei[0], pad])
    dst = jnp.concatenate([ei[1], pad])
    src_sc = src.reshape(NS, CHUNKS, CHUNK)
    dst_sc = dst.reshape(NS, CHUNKS, CHUNK)
    src_es = src.reshape(NC * NS, CHUNKS // 2, CHUNK)
    dst_es = dst.reshape(NC * NS, CHUNKS // 2, CHUNK)
    dst_deg = dst.reshape(NC * NS, EPW)

    x_pad = jnp.pad(x, ((0, NPAD - N), (0, 0)))
    gn_pad = jnp.pad(gnoise, ((0, NPAD - N), (0, 0)))
    wcat = jnp.concatenate([Wmu, Wsig], axis=1)
    bcat = jnp.concatenate([bmu, bsig])

    hist = _deg_kernel()(dst_deg)
    xw = _mm1(x_pad, W1)
    dinv, y1cat = _scale(hist, xw)
    s1cat = _scatter_kernel(False)(y1cat, src_sc, dst_sc)
    y2 = _layer2(s1cat, y1cat, dinv, b1, wcat)
    s2cat = _scatter_kernel(True)(y2, src_es, dst_es)
    z = _zcomp(s2cat, y2, dinv, bcat, gn_pad)
    return _decode(z)


# speedup vs baseline: 12.4890x; 2.0786x over previous
import functools

import jax
import jax.numpy as jnp
from jax import lax
from jax.experimental import pallas as pl
from jax.experimental.pallas import tpu as pltpu
from jax.experimental.pallas import tpu_sc as plsc

N = 10000
NPAD = 10240
IN_DIM = 256
FEAT = 256
LAT = 64
E = 160000
EPAD = 163840

NC, NS, L = 2, 16, 16
CHUNK = 128
CHUNKS = EPAD // NS // CHUNK
RPT = NPAD // NS
EPW = EPAD // (NC * NS)


def _mesh():
    return plsc.VectorSubcoreMesh(
        core_axis_name="c", subcore_axis_name="s",
        num_cores=NC, num_subcores=NS)


def _deg_body(dst_hbm, hist_hbm, dst_v, hist_v):
    c = lax.axis_index("c")
    s = lax.axis_index("s")
    w = s * NC + c
    pltpu.sync_copy(dst_hbm.at[w], dst_v)
    zeros = jnp.zeros((L,), jnp.float32)
    ones = jnp.ones((L,), jnp.float32)

    @pl.loop(0, NPAD // L)
    def _zero(i):
        hist_v[pl.ds(i * L, L)] = zeros

    @pl.loop(0, EPW // L)
    def _count(i):
        idx = dst_v[pl.ds(i * L, L)]
        plsc.addupdate_scatter(hist_v, [idx], ones)

    pltpu.sync_copy(hist_v, hist_hbm.at[w])


@functools.cache
def _deg_kernel():
    return functools.partial(
        pl.kernel,
        out_type=jax.ShapeDtypeStruct((NC * NS, NPAD), jnp.float32),
        mesh=_mesh(),
        scratch_types=[
            pltpu.VMEM((EPW,), jnp.int32),
            pltpu.VMEM((NPAD,), jnp.float32),
        ],
        compiler_params=pltpu.CompilerParams(needs_layout_passes=False),
    )(_deg_body)


def _make_scatter(edge_split):
    DH = 128
    chunks = CHUNKS // 2 if edge_split else CHUNKS

    def body(table_hbm, src_hbm, dst_hbm, out_hbm,
             src_a, src_b, dst_v, rows_a, rows_b,
             sem_ga, sem_gb, sem_ia, sem_ib, acc):
        c = lax.axis_index("c")
        s = lax.axis_index("s")
        w = c * NS + s if edge_split else s
        pltpu.sync_copy(dst_hbm.at[w], dst_v)

        zeros = jnp.zeros((L,), jnp.float32)

        @pl.loop(0, CHUNK)
        def _zr(r):
            @pl.loop(0, DH // L)
            def _zc(k):
                rows_a[r, pl.ds(k * L, L)] = zeros

        for k in range(RPT // CHUNK):
            pltpu.sync_copy(rows_a, acc.at[pl.ds(s * RPT + k * CHUNK, CHUNK)])

        table = table_hbm if edge_split else table_hbm.at[c]
        last = chunks - 1

        def gather(sbuf, rbuf, sem):
            pltpu.async_copy(table.at[sbuf], rbuf, sem)

        def gwait(rbuf, sem):
            pltpu.make_async_copy(table.at[src_a], rbuf, sem).wait()

        def iload(jj, sbuf, sem):
            pltpu.async_copy(src_hbm.at[w, jj], sbuf, sem)

        def iwait(sbuf, sem):
            pltpu.make_async_copy(src_hbm.at[w, 0], sbuf, sem).wait()

        pltpu.sync_copy(src_hbm.at[w, 0], src_a)
        gather(src_a, rows_a, sem_ga)
        iload(1, src_b, sem_ib)
        iwait(src_b, sem_ib)
        plsc.subcore_barrier()

        @pl.loop(0, chunks, step=2)
        def _edge(j):
            gwait(rows_a, sem_ga)
            gather(src_b, rows_b, sem_gb)
            iload(jnp.minimum(j + 2, last), src_a, sem_ia)
            pltpu.sync_copy(rows_a, acc.at[dst_v.at[j]], add=True)
            iwait(src_a, sem_ia)
            gwait(rows_b, sem_gb)
            gather(src_a, rows_a, sem_ga)
            iload(jnp.minimum(j + 3, last), src_b, sem_ib)
            pltpu.sync_copy(rows_b, acc.at[dst_v.at[j + 1]], add=True)
            iwait(src_b, sem_ib)

        gwait(rows_a, sem_ga)
        plsc.subcore_barrier()
        pltpu.sync_copy(acc.at[pl.ds(s * RPT, RPT)],
                        out_hbm.at[c, pl.ds(s * RPT, RPT)])

    tshape = (NPAD, DH) if edge_split else (NC, NPAD, DH)
    nw = NC * NS if edge_split else NS
    return functools.partial(
        pl.kernel,
        out_type=jax.ShapeDtypeStruct((NC, NPAD, DH), jnp.float32),
        mesh=_mesh(),
        scratch_types=[
            pltpu.VMEM((CHUNK,), jnp.int32),
            pltpu.VMEM((CHUNK,), jnp.int32),
            pltpu.VMEM((chunks, CHUNK), jnp.int32),
            pltpu.VMEM((CHUNK, DH), jnp.float32),
            pltpu.VMEM((CHUNK, DH), jnp.float32),
            pltpu.SemaphoreType.DMA,
            pltpu.SemaphoreType.DMA,
            pltpu.SemaphoreType.DMA,
            pltpu.SemaphoreType.DMA,
            pltpu.VMEM_SHARED((NPAD, DH), jnp.float32),
        ],
        compiler_params=pltpu.CompilerParams(needs_layout_passes=False),
    )(body)


_scatter_kernel = functools.cache(_make_scatter)


_BM = 512


def _layer1_body(hist_ref, x_ref, w_ref, dinv_ref, ycat_ref):
    deg = jnp.sum(hist_ref[...], axis=0) + 1.0
    dinv = lax.rsqrt(jnp.maximum(deg, 1.0))
    xw = jnp.dot(x_ref[...], w_ref[...], preferred_element_type=jnp.float32)
    y = xw * dinv[:, None]
    dinv_ref[...] = dinv
    ycat_ref[0] = y[:, : FEAT // 2]
    ycat_ref[1] = y[:, FEAT // 2:]


def _layer1(hist, x, w):
    return pl.pallas_call(
        _layer1_body,
        grid=(NPAD // _BM,),
        in_specs=[pl.BlockSpec((NC * NS, _BM), lambda i: (0, i)),
                  pl.BlockSpec((_BM, IN_DIM), lambda i: (i, 0)),
                  pl.BlockSpec((IN_DIM, FEAT), lambda i: (0, 0))],
        out_specs=[pl.BlockSpec((_BM,), lambda i: (i,)),
                   pl.BlockSpec((NC, _BM, FEAT // 2), lambda i: (0, i, 0))],
        out_shape=[jax.ShapeDtypeStruct((NPAD,), jnp.float32),
                   jax.ShapeDtypeStruct((NC, NPAD, FEAT // 2), jnp.float32)],
    )(hist, x, w)


def _layer2_body(s1_ref, y1_ref, dinv_ref, b1_ref, wcat_ref, y2_ref, i_ref=None):
    i = pl.program_id(0)
    s = jnp.concatenate([s1_ref[0], s1_ref[1]], axis=1)
    y1 = jnp.concatenate([y1_ref[0], y1_ref[1]], axis=1)
    dinv = dinv_ref[...]
    h = jnp.maximum(dinv[:, None] * (s + y1) + b1_ref[...][None, :], 0.0)
    row = i * _BM + lax.broadcasted_iota(jnp.int32, (_BM, 1), 0)
    h = jnp.where(row < N, h, 0.0)
    c = jnp.dot(h, wcat_ref[...], preferred_element_type=jnp.float32)
    y2_ref[...] = c * dinv[:, None]


def _layer2(s1cat, y1cat, dinv, b1, wcat):
    return pl.pallas_call(
        _layer2_body,
        grid=(NPAD // _BM,),
        in_specs=[pl.BlockSpec((NC, _BM, FEAT // 2), lambda i: (0, i, 0)),
                  pl.BlockSpec((NC, _BM, FEAT // 2), lambda i: (0, i, 0)),
                  pl.BlockSpec((_BM,), lambda i: (i,)),
                  pl.BlockSpec((FEAT,), lambda i: (0,)),
                  pl.BlockSpec((FEAT, 2 * LAT), lambda i: (0, 0))],
        out_specs=pl.BlockSpec((_BM, 2 * LAT), lambda i: (i, 0)),
        out_shape=jax.ShapeDtypeStruct((NPAD, 2 * LAT), jnp.float32),
    )(s1cat, y1cat, dinv, b1, wcat)


def _zcomp_body(s2_ref, y2_ref, dinv_ref, bcat_ref, gn_ref, z_ref):
    s = s2_ref[0] + s2_ref[1]
    y2 = y2_ref[...]
    o = dinv_ref[...][:, None] * (s + y2) + bcat_ref[...][None, :]
    xu = o[:, :LAT]
    xs = o[:, LAT:]
    z_ref[...] = gn_ref[...] * jnp.exp(xs) + xu


def _zcomp(s2cat, y2cat, dinv, bcat, gn):
    return pl.pallas_call(
        _zcomp_body,
        grid=(NPAD // _BM,),
        in_specs=[pl.BlockSpec((NC, _BM, 2 * LAT), lambda i: (0, i, 0)),
                  pl.BlockSpec((_BM, 2 * LAT), lambda i: (i, 0)),
                  pl.BlockSpec((_BM,), lambda i: (i,)),
                  pl.BlockSpec((2 * LAT,), lambda i: (0,)),
                  pl.BlockSpec((_BM, LAT), lambda i: (i, 0))],
        out_specs=pl.BlockSpec((_BM, LAT), lambda i: (i, 0)),
        out_shape=jax.ShapeDtypeStruct((NPAD, LAT), jnp.float32),
    )(s2cat, y2cat, dinv, bcat, gn)


_BD = 512
_BDN = 1024


def _decode_body(zr_ref, zc_ref, o_ref):
    p = lax.dot_general(zr_ref[...], zc_ref[...],
                        (((1,), (1,)), ((), ())),
                        preferred_element_type=jnp.float32)
    o_ref[...] = 0.5 * jnp.tanh(0.5 * p) + 0.5


def _decode(z):
    return pl.pallas_call(
        _decode_body,
        grid=(NPAD // _BD, NPAD // _BDN),
        in_specs=[pl.BlockSpec((_BD, LAT), lambda i, j: (i, 0)),
                  pl.BlockSpec((_BDN, LAT), lambda i, j: (j, 0))],
        out_specs=pl.BlockSpec((_BD, _BDN), lambda i, j: (i, j)),
        out_shape=jax.ShapeDtypeStruct((N, N), jnp.float32),
    )(z, z)


def kernel(x, edge_index, W1, b1, Wmu, bmu, Wsig, bsig, gnoise):
    ei = edge_index.astype(jnp.int32)
    pad = N + jnp.arange(EPAD - E, dtype=jnp.int32) % (NPAD - N)
    src = jnp.concatenate([ei[0], pad])
    dst = jnp.concatenate([ei[1], pad])
    src_sc = src.reshape(NS, CHUNKS, CHUNK)
    dst_sc = dst.reshape(NS, CHUNKS, CHUNK)
    src_es = src.reshape(NC * NS, CHUNKS // 2, CHUNK)
    dst_es = dst.reshape(NC * NS, CHUNKS // 2, CHUNK)
    dst_deg = dst.reshape(NC * NS, EPW)

    x_pad = jnp.pad(x, ((0, NPAD - N), (0, 0)))
    gn_pad = jnp.pad(gnoise, ((0, NPAD - N), (0, 0)))
    wcat = jnp.concatenate([Wmu, Wsig], axis=1)
    bcat = jnp.concatenate([bmu, bsig])

    hist = _deg_kernel()(dst_deg)
    dinv, y1cat = _layer1(hist, x_pad, W1)
    s1cat = _scatter_kernel(False)(y1cat, src_sc, dst_sc)
    y2 = _layer2(s1cat, y1cat, dinv, b1, wcat)
    s2cat = _scatter_kernel(True)(y2, src_es, dst_es)
    z = _zcomp(s2cat, y2, dinv, bcat, gn_pad)
    return _decode(z)


# speedup vs baseline: 14.5676x; 1.1664x over previous
import functools

import jax
import jax.numpy as jnp
from jax import lax
from jax.experimental import pallas as pl
from jax.experimental.pallas import tpu as pltpu
from jax.experimental.pallas import tpu_sc as plsc

N = 10000
NPAD = 10240
IN_DIM = 256
FEAT = 256
LAT = 64
E = 160000
EPAD = 163840

NC, NS, L = 2, 16, 16
CHUNK = 128
CHUNKS = EPAD // NS // CHUNK
RPT = NPAD // NS
EPW = EPAD // (NC * NS)


def _mesh():
    return plsc.VectorSubcoreMesh(
        core_axis_name="c", subcore_axis_name="s",
        num_cores=NC, num_subcores=NS)


def _deg_body(dst_hbm, hist_hbm, dst_v, hist_v):
    c = lax.axis_index("c")
    s = lax.axis_index("s")
    w = s * NC + c
    pltpu.sync_copy(dst_hbm.at[w], dst_v)
    zeros = jnp.zeros((L,), jnp.float32)
    ones = jnp.ones((L,), jnp.float32)

    @pl.loop(0, NPAD // L)
    def _zero(i):
        hist_v[pl.ds(i * L, L)] = zeros

    @pl.loop(0, EPW // L)
    def _count(i):
        idx = dst_v[pl.ds(i * L, L)]
        plsc.addupdate_scatter(hist_v, [idx], ones)

    pltpu.sync_copy(hist_v, hist_hbm.at[w])


@functools.cache
def _deg_kernel():
    return functools.partial(
        pl.kernel,
        out_type=jax.ShapeDtypeStruct((NC * NS, NPAD), jnp.float32),
        mesh=_mesh(),
        scratch_types=[
            pltpu.VMEM((EPW,), jnp.int32),
            pltpu.VMEM((NPAD,), jnp.float32),
        ],
        compiler_params=pltpu.CompilerParams(needs_layout_passes=False),
    )(_deg_body)


def _make_scatter(edge_split):
    DH = 128
    chunks = CHUNKS // 2 if edge_split else CHUNKS

    def body(table_hbm, src_hbm, dst_hbm, out_hbm,
             src_a, src_b, dst_v, rows_a, rows_b,
             sem_ga, sem_gb, sem_ia, sem_ib, acc):
        c = lax.axis_index("c")
        s = lax.axis_index("s")
        w = c * NS + s if edge_split else s
        pltpu.sync_copy(dst_hbm.at[w], dst_v)

        zeros = jnp.zeros((L,), jnp.float32)

        @pl.loop(0, CHUNK)
        def _zr(r):
            @pl.loop(0, DH // L)
            def _zc(k):
                rows_a[r, pl.ds(k * L, L)] = zeros

        for k in range(RPT // CHUNK):
            pltpu.sync_copy(rows_a, acc.at[pl.ds(s * RPT + k * CHUNK, CHUNK)])

        table = table_hbm if edge_split else table_hbm.at[c]
        last = chunks - 1

        def gather(sbuf, rbuf, sem):
            pltpu.async_copy(table.at[sbuf], rbuf, sem)

        def gwait(rbuf, sem):
            pltpu.make_async_copy(table.at[src_a], rbuf, sem).wait()

        def iload(jj, sbuf, sem):
            pltpu.async_copy(src_hbm.at[w, jj], sbuf, sem)

        def iwait(sbuf, sem):
            pltpu.make_async_copy(src_hbm.at[w, 0], sbuf, sem).wait()

        pltpu.sync_copy(src_hbm.at[w, 0], src_a)
        gather(src_a, rows_a, sem_ga)
        iload(1, src_b, sem_ib)
        iwait(src_b, sem_ib)
        plsc.subcore_barrier()

        @pl.loop(0, chunks, step=2)
        def _edge(j):
            gwait(rows_a, sem_ga)
            gather(src_b, rows_b, sem_gb)
            iload(jnp.minimum(j + 2, last), src_a, sem_ia)
            pltpu.sync_copy(rows_a, acc.at[dst_v.at[j]], add=True)
            iwait(src_a, sem_ia)
            gwait(rows_b, sem_gb)
            gather(src_a, rows_a, sem_ga)
            iload(jnp.minimum(j + 3, last), src_b, sem_ib)
            pltpu.sync_copy(rows_b, acc.at[dst_v.at[j + 1]], add=True)
            iwait(src_b, sem_ib)

        gwait(rows_a, sem_ga)
        plsc.subcore_barrier()
        pltpu.sync_copy(acc.at[pl.ds(s * RPT, RPT)],
                        out_hbm.at[c, pl.ds(s * RPT, RPT)])

    tshape = (NPAD, DH) if edge_split else (NC, NPAD, DH)
    nw = NC * NS if edge_split else NS
    return functools.partial(
        pl.kernel,
        out_type=jax.ShapeDtypeStruct((NC, NPAD, DH), jnp.float32),
        mesh=_mesh(),
        scratch_types=[
            pltpu.VMEM((CHUNK,), jnp.int32),
            pltpu.VMEM((CHUNK,), jnp.int32),
            pltpu.VMEM((chunks, CHUNK), jnp.int32),
            pltpu.VMEM((CHUNK, DH), jnp.float32),
            pltpu.VMEM((CHUNK, DH), jnp.float32),
            pltpu.SemaphoreType.DMA,
            pltpu.SemaphoreType.DMA,
            pltpu.SemaphoreType.DMA,
            pltpu.SemaphoreType.DMA,
            pltpu.VMEM_SHARED((NPAD, DH), jnp.float32),
        ],
        compiler_params=pltpu.CompilerParams(needs_layout_passes=False),
    )(body)


_scatter_kernel = functools.cache(_make_scatter)


_BM = 512


def _layer1_body(hist_ref, x_ref, w_ref, dinv_ref, ycat_ref):
    deg = jnp.sum(hist_ref[...], axis=0) + 1.0
    dinv = lax.rsqrt(jnp.maximum(deg, 1.0))
    xw = jnp.dot(x_ref[...], w_ref[...], preferred_element_type=jnp.float32)
    y = xw * dinv[:, None]
    dinv_ref[...] = dinv
    ycat_ref[0] = y[:, : FEAT // 2]
    ycat_ref[1] = y[:, FEAT // 2:]


def _layer1(hist, x, w):
    return pl.pallas_call(
        _layer1_body,
        grid=(NPAD // _BM,),
        in_specs=[pl.BlockSpec((NC * NS, _BM), lambda i: (0, i)),
                  pl.BlockSpec((_BM, IN_DIM), lambda i: (i, 0)),
                  pl.BlockSpec((IN_DIM, FEAT), lambda i: (0, 0))],
        out_specs=[pl.BlockSpec((_BM,), lambda i: (i,)),
                   pl.BlockSpec((NC, _BM, FEAT // 2), lambda i: (0, i, 0))],
        out_shape=[jax.ShapeDtypeStruct((NPAD,), jnp.float32),
                   jax.ShapeDtypeStruct((NC, NPAD, FEAT // 2), jnp.float32)],
    )(hist, x, w)


def _layer2_body(s1_ref, y1_ref, dinv_ref, b1_ref, wcat_ref, y2_ref, i_ref=None):
    i = pl.program_id(0)
    s = jnp.concatenate([s1_ref[0], s1_ref[1]], axis=1)
    y1 = jnp.concatenate([y1_ref[0], y1_ref[1]], axis=1)
    dinv = dinv_ref[...]
    h = jnp.maximum(dinv[:, None] * (s + y1) + b1_ref[...][None, :], 0.0)
    row = i * _BM + lax.broadcasted_iota(jnp.int32, (_BM, 1), 0)
    h = jnp.where(row < N, h, 0.0)
    c = jnp.dot(h, wcat_ref[...], preferred_element_type=jnp.float32)
    y2_ref[...] = c * dinv[:, None]


def _layer2(s1cat, y1cat, dinv, b1, wcat):
    return pl.pallas_call(
        _layer2_body,
        grid=(NPAD // _BM,),
        in_specs=[pl.BlockSpec((NC, _BM, FEAT // 2), lambda i: (0, i, 0)),
                  pl.BlockSpec((NC, _BM, FEAT // 2), lambda i: (0, i, 0)),
                  pl.BlockSpec((_BM,), lambda i: (i,)),
                  pl.BlockSpec((FEAT,), lambda i: (0,)),
                  pl.BlockSpec((FEAT, 2 * LAT), lambda i: (0, 0))],
        out_specs=pl.BlockSpec((_BM, 2 * LAT), lambda i: (i, 0)),
        out_shape=jax.ShapeDtypeStruct((NPAD, 2 * LAT), jnp.float32),
    )(s1cat, y1cat, dinv, b1, wcat)


def _zcomp_body(s2_ref, y2_ref, dinv_ref, bcat_ref, gn_ref, z_ref):
    s = s2_ref[0] + s2_ref[1]
    y2 = y2_ref[...]
    o = dinv_ref[...][:, None] * (s + y2) + bcat_ref[...][None, :]
    xu = o[:, :LAT]
    xs = o[:, LAT:]
    z = gn_ref[...] * jnp.exp(xs) + xu
    z_ref[...] = (z * 0.7071067811865476).astype(jnp.bfloat16)


def _zcomp(s2cat, y2cat, dinv, bcat, gn):
    return pl.pallas_call(
        _zcomp_body,
        grid=(NPAD // _BM,),
        in_specs=[pl.BlockSpec((NC, _BM, 2 * LAT), lambda i: (0, i, 0)),
                  pl.BlockSpec((_BM, 2 * LAT), lambda i: (i, 0)),
                  pl.BlockSpec((_BM,), lambda i: (i,)),
                  pl.BlockSpec((2 * LAT,), lambda i: (0,)),
                  pl.BlockSpec((_BM, LAT), lambda i: (i, 0))],
        out_specs=pl.BlockSpec((_BM, LAT), lambda i: (i, 0)),
        out_shape=jax.ShapeDtypeStruct((NPAD, LAT), jnp.bfloat16),
    )(s2cat, y2cat, dinv, bcat, gn)


_BD = 512
_BDN = 2048


def _decode_body(zr_ref, zc_ref, o_ref):
    p = lax.dot_general(zr_ref[...], zc_ref[...],
                        (((1,), (1,)), ((), ())),
                        preferred_element_type=jnp.float32)
    o_ref[...] = 0.5 * jnp.tanh(p) + 0.5


def _decode(z):
    return pl.pallas_call(
        _decode_body,
        grid=(NPAD // _BD, NPAD // _BDN),
        in_specs=[pl.BlockSpec((_BD, LAT), lambda i, j: (i, 0)),
                  pl.BlockSpec((_BDN, LAT), lambda i, j: (j, 0))],
        out_specs=pl.BlockSpec((_BD, _BDN), lambda i, j: (i, j)),
        out_shape=jax.ShapeDtypeStruct((N, N), jnp.float32),
    )(z, z)


def kernel(x, edge_index, W1, b1, Wmu, bmu, Wsig, bsig, gnoise):
    ei = edge_index.astype(jnp.int32)
    pad = N + jnp.arange(EPAD - E, dtype=jnp.int32) % (NPAD - N)
    src = jnp.concatenate([ei[0], pad])
    dst = jnp.concatenate([ei[1], pad])
    src_sc = src.reshape(NS, CHUNKS, CHUNK)
    dst_sc = dst.reshape(NS, CHUNKS, CHUNK)
    src_es = src.reshape(NC * NS, CHUNKS // 2, CHUNK)
    dst_es = dst.reshape(NC * NS, CHUNKS // 2, CHUNK)
    dst_deg = dst.reshape(NC * NS, EPW)

    x_pad = jnp.pad(x, ((0, NPAD - N), (0, 0)))
    gn_pad = jnp.pad(gnoise, ((0, NPAD - N), (0, 0)))
    wcat = jnp.concatenate([Wmu, Wsig], axis=1)
    bcat = jnp.concatenate([bmu, bsig])

    hist = _deg_kernel()(dst_deg)
    dinv, y1cat = _layer1(hist, x_pad, W1)
    s1cat = _scatter_kernel(False)(y1cat, src_sc, dst_sc)
    y2 = _layer2(s1cat, y1cat, dinv, b1, wcat)
    s2cat = _scatter_kernel(True)(y2, src_es, dst_es)
    z = _zcomp(s2cat, y2, dinv, bcat, gn_pad)
    return _decode(z)


# speedup vs baseline: 15.7736x; 1.0828x over previous
import functools

import jax
import jax.numpy as jnp
from jax import lax
from jax.experimental import pallas as pl
from jax.experimental.pallas import tpu as pltpu
from jax.experimental.pallas import tpu_sc as plsc

N = 10000
NPAD = 10240
IN_DIM = 256
FEAT = 256
LAT = 64
E = 160000
EPAD = 163840

NC, NS, L = 2, 16, 16
CHUNK = 128
CHUNKS = EPAD // NS // CHUNK
RPT = NPAD // NS
EPW = EPAD // (NC * NS)


def _mesh():
    return plsc.VectorSubcoreMesh(
        core_axis_name="c", subcore_axis_name="s",
        num_cores=NC, num_subcores=NS)


def _deg_body(dst_hbm, hist_hbm, dst_v, hist_v):
    c = lax.axis_index("c")
    s = lax.axis_index("s")
    w = s * NC + c
    pltpu.sync_copy(dst_hbm.at[w], dst_v)
    zeros = jnp.zeros((L,), jnp.float32)
    ones = jnp.ones((L,), jnp.float32)

    @pl.loop(0, NPAD // L)
    def _zero(i):
        hist_v[pl.ds(i * L, L)] = zeros

    @pl.loop(0, EPW // L)
    def _count(i):
        idx = dst_v[pl.ds(i * L, L)]
        plsc.addupdate_scatter(hist_v, [idx], ones)

    pltpu.sync_copy(hist_v, hist_hbm.at[w])


@functools.cache
def _deg_kernel():
    return functools.partial(
        pl.kernel,
        out_type=jax.ShapeDtypeStruct((NC * NS, NPAD), jnp.float32),
        mesh=_mesh(),
        scratch_types=[
            pltpu.VMEM((EPW,), jnp.int32),
            pltpu.VMEM((NPAD,), jnp.float32),
        ],
        compiler_params=pltpu.CompilerParams(needs_layout_passes=False),
    )(_deg_body)


def _make_scatter(edge_split):
    DH = 128
    chunks = CHUNKS // 2 if edge_split else CHUNKS

    def body(table_hbm, src_hbm, dst_hbm, out_hbm,
             src_a, src_b, dst_v, rows_a, rows_b,
             sem_ga, sem_gb, sem_ia, sem_ib, acc):
        c = lax.axis_index("c")
        s = lax.axis_index("s")
        w = c * NS + s if edge_split else s
        pltpu.sync_copy(dst_hbm.at[w], dst_v)

        zeros = jnp.zeros((L,), jnp.float32)

        @pl.loop(0, CHUNK)
        def _zr(r):
            @pl.loop(0, DH // L)
            def _zc(k):
                rows_a[r, pl.ds(k * L, L)] = zeros

        for k in range(RPT // CHUNK):
            pltpu.sync_copy(rows_a, acc.at[pl.ds(s * RPT + k * CHUNK, CHUNK)])

        table = table_hbm if edge_split else table_hbm.at[c]
        last = chunks - 1

        def gather(sbuf, rbuf, sem):
            pltpu.async_copy(table.at[sbuf], rbuf, sem)

        def gwait(rbuf, sem):
            pltpu.make_async_copy(table.at[src_a], rbuf, sem).wait()

        def iload(jj, sbuf, sem):
            pltpu.async_copy(src_hbm.at[w, jj], sbuf, sem)

        def iwait(sbuf, sem):
            pltpu.make_async_copy(src_hbm.at[w, 0], sbuf, sem).wait()

        pltpu.sync_copy(src_hbm.at[w, 0], src_a)
        gather(src_a, rows_a, sem_ga)
        iload(1, src_b, sem_ib)
        iwait(src_b, sem_ib)
        plsc.subcore_barrier()

        @pl.loop(0, chunks, step=2)
        def _edge(j):
            gwait(rows_a, sem_ga)
            gather(src_b, rows_b, sem_gb)
            iload(jnp.minimum(j + 2, last), src_a, sem_ia)
            pltpu.sync_copy(rows_a, acc.at[dst_v.at[j]], add=True)
            iwait(src_a, sem_ia)
            gwait(rows_b, sem_gb)
            gather(src_a, rows_a, sem_ga)
            iload(jnp.minimum(j + 3, last), src_b, sem_ib)
            pltpu.sync_copy(rows_b, acc.at[dst_v.at[j + 1]], add=True)
            iwait(src_b, sem_ib)

        gwait(rows_a, sem_ga)
        plsc.subcore_barrier()
        pltpu.sync_copy(acc.at[pl.ds(s * RPT, RPT)],
                        out_hbm.at[c, pl.ds(s * RPT, RPT)])

    tshape = (NPAD, DH) if edge_split else (NC, NPAD, DH)
    nw = NC * NS if edge_split else NS
    return functools.partial(
        pl.kernel,
        out_type=jax.ShapeDtypeStruct((NC, NPAD, DH), jnp.float32),
        mesh=_mesh(),
        scratch_types=[
            pltpu.VMEM((CHUNK,), jnp.int32),
            pltpu.VMEM((CHUNK,), jnp.int32),
            pltpu.VMEM((chunks, CHUNK), jnp.int32),
            pltpu.VMEM((CHUNK, DH), jnp.float32),
            pltpu.VMEM((CHUNK, DH), jnp.float32),
            pltpu.SemaphoreType.DMA,
            pltpu.SemaphoreType.DMA,
            pltpu.SemaphoreType.DMA,
            pltpu.SemaphoreType.DMA,
            pltpu.VMEM_SHARED((NPAD, DH), jnp.float32),
        ],
        compiler_params=pltpu.CompilerParams(needs_layout_passes=False),
    )(body)


_scatter_kernel = functools.cache(_make_scatter)


_BM = 512


def _layer1_body(hist_ref, x_ref, w_ref, dinv_ref, ycat_ref):
    i = pl.program_id(0)
    deg = jnp.sum(hist_ref[...], axis=0) + 1.0
    dinv = lax.rsqrt(jnp.maximum(deg, 1.0))
    xw = jnp.dot(x_ref[...], w_ref[...], preferred_element_type=jnp.float32)
    y = xw * dinv[:, None]
    row = i * _BM + lax.broadcasted_iota(jnp.int32, (_BM, 1), 0)
    y = jnp.where(row < N, y, 0.0)
    dinv_ref[...] = dinv
    ycat_ref[0] = y[:, : FEAT // 2]
    ycat_ref[1] = y[:, FEAT // 2:]


def _layer1(hist, x, w):
    return pl.pallas_call(
        _layer1_body,
        grid=(NPAD // _BM,),
        in_specs=[pl.BlockSpec((NC * NS, _BM), lambda i: (0, i)),
                  pl.BlockSpec((_BM, IN_DIM), lambda i: (i, 0)),
                  pl.BlockSpec((IN_DIM, FEAT), lambda i: (0, 0))],
        out_specs=[pl.BlockSpec((_BM,), lambda i: (i,)),
                   pl.BlockSpec((NC, _BM, FEAT // 2), lambda i: (0, i, 0))],
        out_shape=[jax.ShapeDtypeStruct((NPAD,), jnp.float32),
                   jax.ShapeDtypeStruct((NC, NPAD, FEAT // 2), jnp.float32)],
    )(hist, x, w)


def _layer2_body(s1_ref, y1_ref, dinv_ref, b1_ref, wcat_ref, y2_ref, i_ref=None):
    i = pl.program_id(0)
    s = jnp.concatenate([s1_ref[0], s1_ref[1]], axis=1)
    y1 = jnp.concatenate([y1_ref[0], y1_ref[1]], axis=1)
    dinv = dinv_ref[...]
    h = jnp.maximum(dinv[:, None] * (s + y1) + b1_ref[...][None, :], 0.0)
    row = i * _BM + lax.broadcasted_iota(jnp.int32, (_BM, 1), 0)
    h = jnp.where(row < N, h, 0.0)
    c = jnp.dot(h, wcat_ref[...], preferred_element_type=jnp.float32)
    y2_ref[...] = c * dinv[:, None]


def _layer2(s1cat, y1cat, dinv, b1, wcat):
    return pl.pallas_call(
        _layer2_body,
        grid=(NPAD // _BM,),
        in_specs=[pl.BlockSpec((NC, _BM, FEAT // 2), lambda i: (0, i, 0)),
                  pl.BlockSpec((NC, _BM, FEAT // 2), lambda i: (0, i, 0)),
                  pl.BlockSpec((_BM,), lambda i: (i,)),
                  pl.BlockSpec((FEAT,), lambda i: (0,)),
                  pl.BlockSpec((FEAT, 2 * LAT), lambda i: (0, 0))],
        out_specs=pl.BlockSpec((_BM, 2 * LAT), lambda i: (i, 0)),
        out_shape=jax.ShapeDtypeStruct((NPAD, 2 * LAT), jnp.float32),
    )(s1cat, y1cat, dinv, b1, wcat)


def _zcomp_body(s2_ref, y2_ref, dinv_ref, bcat_ref, gn_ref, z_ref):
    s = s2_ref[0] + s2_ref[1]
    y2 = y2_ref[...]
    o = dinv_ref[...][:, None] * (s + y2) + bcat_ref[...][None, :]
    xu = o[:, :LAT]
    xs = o[:, LAT:]
    z = gn_ref[...] * jnp.exp(xs) + xu
    z_ref[...] = (z * 0.7071067811865476).astype(jnp.bfloat16)


def _zcomp(s2cat, y2cat, dinv, bcat, gn):
    return pl.pallas_call(
        _zcomp_body,
        grid=(NPAD // _BM,),
        in_specs=[pl.BlockSpec((NC, _BM, 2 * LAT), lambda i: (0, i, 0)),
                  pl.BlockSpec((_BM, 2 * LAT), lambda i: (i, 0)),
                  pl.BlockSpec((_BM,), lambda i: (i,)),
                  pl.BlockSpec((2 * LAT,), lambda i: (0,)),
                  pl.BlockSpec((_BM, LAT), lambda i: (i, 0))],
        out_specs=pl.BlockSpec((_BM, LAT), lambda i: (i, 0)),
        out_shape=jax.ShapeDtypeStruct((NPAD, LAT), jnp.bfloat16),
    )(s2cat, y2cat, dinv, bcat, gn)


_BD = 1024
_BDN = 2048


def _decode_body(zr_ref, zc_ref, o_ref):
    p = lax.dot_general(zr_ref[...], zc_ref[...],
                        (((1,), (1,)), ((), ())),
                        preferred_element_type=jnp.float32)
    o_ref[...] = 0.5 * jnp.tanh(p) + 0.5


def _decode(z):
    return pl.pallas_call(
        _decode_body,
        grid=(NPAD // _BD, NPAD // _BDN),
        in_specs=[pl.BlockSpec((_BD, LAT), lambda i, j: (i, 0)),
                  pl.BlockSpec((_BDN, LAT), lambda i, j: (j, 0))],
        out_specs=pl.BlockSpec((_BD, _BDN), lambda i, j: (i, j)),
        out_shape=jax.ShapeDtypeStruct((N, N), jnp.float32),
    )(z, z)


def kernel(x, edge_index, W1, b1, Wmu, bmu, Wsig, bsig, gnoise):
    ei = edge_index.astype(jnp.int32)
    pad = N + jnp.arange(EPAD - E, dtype=jnp.int32) % (NPAD - N)
    src = jnp.concatenate([ei[0], pad])
    dst = jnp.concatenate([ei[1], pad])
    src_sc = src.reshape(NS, CHUNKS, CHUNK)
    dst_sc = dst.reshape(NS, CHUNKS, CHUNK)
    src_es = src.reshape(NC * NS, CHUNKS // 2, CHUNK)
    dst_es = dst.reshape(NC * NS, CHUNKS // 2, CHUNK)
    dst_deg = dst.reshape(NC * NS, EPW)

    wcat = jnp.concatenate([Wmu, Wsig], axis=1)
    bcat = jnp.concatenate([bmu, bsig])

    hist = _deg_kernel()(dst_deg)
    dinv, y1cat = _layer1(hist, x, W1)
    s1cat = _scatter_kernel(False)(y1cat, src_sc, dst_sc)
    y2 = _layer2(s1cat, y1cat, dinv, b1, wcat)
    s2cat = _scatter_kernel(True)(y2, src_es, dst_es)
    z = _zcomp(s2cat, y2, dinv, bcat, gnoise)
    return _decode(z)


# speedup vs baseline: 15.8208x; 1.0030x over previous
import functools

import jax
import jax.numpy as jnp
from jax import lax
from jax.experimental import pallas as pl
from jax.experimental.pallas import tpu as pltpu
from jax.experimental.pallas import tpu_sc as plsc

N = 10000
NPAD = 10240
IN_DIM = 256
FEAT = 256
LAT = 64
E = 160000
EPAD = 163840

NC, NS, L = 2, 16, 16
CHUNK = 128
CHUNKS = EPAD // NS // CHUNK
RPT = NPAD // NS
EPW = EPAD // (NC * NS)


def _mesh():
    return plsc.VectorSubcoreMesh(
        core_axis_name="c", subcore_axis_name="s",
        num_cores=NC, num_subcores=NS)


def _deg_body(dst_hbm, hist_hbm, dst_v, hist_v):
    c = lax.axis_index("c")
    s = lax.axis_index("s")
    w = s * NC + c
    pltpu.sync_copy(dst_hbm.at[w], dst_v)
    zeros = jnp.zeros((L,), jnp.float32)
    ones = jnp.ones((L,), jnp.float32)

    @pl.loop(0, NPAD // L)
    def _zero(i):
        hist_v[pl.ds(i * L, L)] = zeros

    @pl.loop(0, EPW // L)
    def _count(i):
        idx = dst_v[pl.ds(i * L, L)]
        plsc.addupdate_scatter(hist_v, [idx], ones)

    pltpu.sync_copy(hist_v, hist_hbm.at[w])


@functools.cache
def _deg_kernel():
    return functools.partial(
        pl.kernel,
        out_type=jax.ShapeDtypeStruct((NC * NS, NPAD), jnp.float32),
        mesh=_mesh(),
        scratch_types=[
            pltpu.VMEM((EPW,), jnp.int32),
            pltpu.VMEM((NPAD,), jnp.float32),
        ],
        compiler_params=pltpu.CompilerParams(needs_layout_passes=False),
    )(_deg_body)


def _make_scatter(edge_split):
    DH = 128
    chunks = CHUNKS // 2 if edge_split else CHUNKS

    def body(table_hbm, src_hbm, dst_hbm, out_hbm,
             src_a, src_b, dst_v, rows_a, rows_b,
             sem_ga, sem_gb, sem_ia, sem_ib, sem_sa, sem_sb, acc):
        c = lax.axis_index("c")
        s = lax.axis_index("s")
        w = c * NS + s if edge_split else s
        pltpu.sync_copy(dst_hbm.at[w], dst_v)

        zeros = jnp.zeros((L,), jnp.float32)

        @pl.loop(0, CHUNK)
        def _zr(r):
            @pl.loop(0, DH // L)
            def _zc(k):
                rows_a[r, pl.ds(k * L, L)] = zeros

        for k in range(RPT // CHUNK):
            pltpu.sync_copy(rows_a, acc.at[pl.ds(s * RPT + k * CHUNK, CHUNK)])

        table = table_hbm if edge_split else table_hbm.at[c]
        last = chunks - 1

        def gather(sbuf, rbuf, sem):
            pltpu.async_copy(table.at[sbuf], rbuf, sem)

        def gwait(rbuf, sem):
            pltpu.make_async_copy(table.at[src_a], rbuf, sem).wait()

        def iload(jj, sbuf, sem):
            pltpu.async_copy(src_hbm.at[w, jj], sbuf, sem)

        def iwait(sbuf, sem):
            pltpu.make_async_copy(src_hbm.at[w, 0], sbuf, sem).wait()

        def scat(rbuf, jj, sem):
            pltpu.async_copy(rbuf, acc.at[dst_v.at[jj]], sem, add=True)

        def swait(rbuf, sem):
            pltpu.make_async_copy(table.at[src_a], rbuf, sem).wait()

        pltpu.sync_copy(src_hbm.at[w, 0], src_a)
        gather(src_a, rows_a, sem_ga)
        iload(1, src_b, sem_ib)
        iwait(src_b, sem_ib)
        plsc.subcore_barrier()

        gwait(rows_a, sem_ga)
        scat(rows_a, 0, sem_sa)
        gather(src_b, rows_b, sem_gb)
        iload(2, src_a, sem_ia)
        gwait(rows_b, sem_gb)
        scat(rows_b, 1, sem_sb)
        swait(rows_a, sem_sa)
        iwait(src_a, sem_ia)
        gather(src_a, rows_a, sem_ga)
        iload(3, src_b, sem_ib)
        iwait(src_b, sem_ib)

        @pl.loop(2, chunks, step=2)
        def _edge(j):
            gwait(rows_a, sem_ga)
            scat(rows_a, j, sem_sa)
            swait(rows_b, sem_sb)
            gather(src_b, rows_b, sem_gb)
            iload(jnp.minimum(j + 2, last), src_a, sem_ia)
            gwait(rows_b, sem_gb)
            scat(rows_b, j + 1, sem_sb)
            swait(rows_a, sem_sa)
            iwait(src_a, sem_ia)
            gather(src_a, rows_a, sem_ga)
            iload(jnp.minimum(j + 3, last), src_b, sem_ib)
            iwait(src_b, sem_ib)

        swait(rows_b, sem_sb)
        gwait(rows_a, sem_ga)
        plsc.subcore_barrier()
        pltpu.sync_copy(acc.at[pl.ds(s * RPT, RPT)],
                        out_hbm.at[c, pl.ds(s * RPT, RPT)])

    tshape = (NPAD, DH) if edge_split else (NC, NPAD, DH)
    nw = NC * NS if edge_split else NS
    return functools.partial(
        pl.kernel,
        out_type=jax.ShapeDtypeStruct((NC, NPAD, DH), jnp.float32),
        mesh=_mesh(),
        scratch_types=[
            pltpu.VMEM((CHUNK,), jnp.int32),
            pltpu.VMEM((CHUNK,), jnp.int32),
            pltpu.VMEM((chunks, CHUNK), jnp.int32),
            pltpu.VMEM((CHUNK, DH), jnp.float32),
            pltpu.VMEM((CHUNK, DH), jnp.float32),
            pltpu.SemaphoreType.DMA,
            pltpu.SemaphoreType.DMA,
            pltpu.SemaphoreType.DMA,
            pltpu.SemaphoreType.DMA,
            pltpu.SemaphoreType.DMA,
            pltpu.SemaphoreType.DMA,
            pltpu.VMEM_SHARED((NPAD, DH), jnp.float32),
        ],
        compiler_params=pltpu.CompilerParams(needs_layout_passes=False),
    )(body)


_scatter_kernel = functools.cache(_make_scatter)


_BM = 512


def _layer1_body(hist_ref, x_ref, w_ref, dinv_ref, ycat_ref):
    i = pl.program_id(0)
    deg = jnp.sum(hist_ref[...], axis=0) + 1.0
    dinv = lax.rsqrt(jnp.maximum(deg, 1.0))
    xw = jnp.dot(x_ref[...], w_ref[...], preferred_element_type=jnp.float32)
    y = xw * dinv[:, None]
    row = i * _BM + lax.broadcasted_iota(jnp.int32, (_BM, 1), 0)
    y = jnp.where(row < N, y, 0.0)
    dinv_ref[...] = dinv
    ycat_ref[0] = y[:, : FEAT // 2]
    ycat_ref[1] = y[:, FEAT // 2:]


def _layer1(hist, x, w):
    return pl.pallas_call(
        _layer1_body,
        grid=(NPAD // _BM,),
        in_specs=[pl.BlockSpec((NC * NS, _BM), lambda i: (0, i)),
                  pl.BlockSpec((_BM, IN_DIM), lambda i: (i, 0)),
                  pl.BlockSpec((IN_DIM, FEAT), lambda i: (0, 0))],
        out_specs=[pl.BlockSpec((_BM,), lambda i: (i,)),
                   pl.BlockSpec((NC, _BM, FEAT // 2), lambda i: (0, i, 0))],
        out_shape=[jax.ShapeDtypeStruct((NPAD,), jnp.float32),
                   jax.ShapeDtypeStruct((NC, NPAD, FEAT // 2), jnp.float32)],
    )(hist, x, w)


def _layer2_body(s1_ref, y1_ref, dinv_ref, b1_ref, wcat_ref, y2_ref, i_ref=None):
    i = pl.program_id(0)
    s = jnp.concatenate([s1_ref[0], s1_ref[1]], axis=1)
    y1 = jnp.concatenate([y1_ref[0], y1_ref[1]], axis=1)
    dinv = dinv_ref[...]
    h = jnp.maximum(dinv[:, None] * (s + y1) + b1_ref[...][None, :], 0.0)
    row = i * _BM + lax.broadcasted_iota(jnp.int32, (_BM, 1), 0)
    h = jnp.where(row < N, h, 0.0)
    c = jnp.dot(h, wcat_ref[...], preferred_element_type=jnp.float32)
    y2_ref[...] = c * dinv[:, None]


def _layer2(s1cat, y1cat, dinv, b1, wcat):
    return pl.pallas_call(
        _layer2_body,
        grid=(NPAD // _BM,),
        in_specs=[pl.BlockSpec((NC, _BM, FEAT // 2), lambda i: (0, i, 0)),
                  pl.BlockSpec((NC, _BM, FEAT // 2), lambda i: (0, i, 0)),
                  pl.BlockSpec((_BM,), lambda i: (i,)),
                  pl.BlockSpec((FEAT,), lambda i: (0,)),
                  pl.BlockSpec((FEAT, 2 * LAT), lambda i: (0, 0))],
        out_specs=pl.BlockSpec((_BM, 2 * LAT), lambda i: (i, 0)),
        out_shape=jax.ShapeDtypeStruct((NPAD, 2 * LAT), jnp.float32),
    )(s1cat, y1cat, dinv, b1, wcat)


def _zcomp_body(s2_ref, y2_ref, dinv_ref, bcat_ref, gn_ref, z_ref):
    s = s2_ref[0] + s2_ref[1]
    y2 = y2_ref[...]
    o = dinv_ref[...][:, None] * (s + y2) + bcat_ref[...][None, :]
    xu = o[:, :LAT]
    xs = o[:, LAT:]
    z = gn_ref[...] * jnp.exp(xs) + xu
    z_ref[...] = (z * 0.7071067811865476).astype(jnp.bfloat16)


def _zcomp(s2cat, y2cat, dinv, bcat, gn):
    return pl.pallas_call(
        _zcomp_body,
        grid=(NPAD // _BM,),
        in_specs=[pl.BlockSpec((NC, _BM, 2 * LAT), lambda i: (0, i, 0)),
                  pl.BlockSpec((_BM, 2 * LAT), lambda i: (i, 0)),
                  pl.BlockSpec((_BM,), lambda i: (i,)),
                  pl.BlockSpec((2 * LAT,), lambda i: (0,)),
                  pl.BlockSpec((_BM, LAT), lambda i: (i, 0))],
        out_specs=pl.BlockSpec((_BM, LAT), lambda i: (i, 0)),
        out_shape=jax.ShapeDtypeStruct((NPAD, LAT), jnp.bfloat16),
    )(s2cat, y2cat, dinv, bcat, gn)


_BD = 1024
_BDN = 2048


def _decode_body(zr_ref, zc_ref, o_ref):
    p = lax.dot_general(zr_ref[...], zc_ref[...],
                        (((1,), (1,)), ((), ())),
                        preferred_element_type=jnp.float32)
    o_ref[...] = 0.5 * jnp.tanh(p) + 0.5


def _decode(z):
    return pl.pallas_call(
        _decode_body,
        grid=(NPAD // _BD, NPAD // _BDN),
        in_specs=[pl.BlockSpec((_BD, LAT), lambda i, j: (i, 0)),
                  pl.BlockSpec((_BDN, LAT), lambda i, j: (j, 0))],
        out_specs=pl.BlockSpec((_BD, _BDN), lambda i, j: (i, j)),
        out_shape=jax.ShapeDtypeStruct((N, N), jnp.float32),
    )(z, z)


def kernel(x, edge_index, W1, b1, Wmu, bmu, Wsig, bsig, gnoise):
    ei = edge_index.astype(jnp.int32)
    pad = N + jnp.arange(EPAD - E, dtype=jnp.int32) % (NPAD - N)
    src = jnp.concatenate([ei[0], pad])
    dst = jnp.concatenate([ei[1], pad])
    src_sc = src.reshape(NS, CHUNKS, CHUNK)
    dst_sc = dst.reshape(NS, CHUNKS, CHUNK)
    src_es = src.reshape(NC * NS, CHUNKS // 2, CHUNK)
    dst_es = dst.reshape(NC * NS, CHUNKS // 2, CHUNK)
    dst_deg = dst.reshape(NC * NS, EPW)

    wcat = jnp.concatenate([Wmu, Wsig], axis=1)
    bcat = jnp.concatenate([bmu, bsig])

    hist = _deg_kernel()(dst_deg)
    dinv, y1cat = _layer1(hist, x, W1)
    s1cat = _scatter_kernel(False)(y1cat, src_sc, dst_sc)
    y2 = _layer2(s1cat, y1cat, dinv, b1, wcat)
    s2cat = _scatter_kernel(True)(y2, src_es, dst_es)
    z = _zcomp(s2cat, y2, dinv, bcat, gnoise)
    return _decode(z)


# speedup vs baseline: 15.9696x; 1.0094x over previous
import functools

import jax
import jax.numpy as jnp
from jax import lax
from jax.experimental import pallas as pl
from jax.experimental.pallas import tpu as pltpu
from jax.experimental.pallas import tpu_sc as plsc

N = 10000
NPAD = 10240
IN_DIM = 256
FEAT = 256
LAT = 64
E = 160000
EPAD = 163840

NC, NS, L = 2, 16, 16
CHUNK = 128
CHUNKS = EPAD // NS // CHUNK
RPT = NPAD // NS
EPW = EPAD // (NC * NS)


def _mesh():
    return plsc.VectorSubcoreMesh(
        core_axis_name="c", subcore_axis_name="s",
        num_cores=NC, num_subcores=NS)


def _deg_body(dst_hbm, hist_hbm, dst_v, hist_v):
    c = lax.axis_index("c")
    s = lax.axis_index("s")
    w = s * NC + c
    pltpu.sync_copy(dst_hbm.at[w], dst_v)
    zeros = jnp.zeros((L,), jnp.float32)
    ones = jnp.ones((L,), jnp.float32)

    @pl.loop(0, NPAD // L)
    def _zero(i):
        hist_v[pl.ds(i * L, L)] = zeros

    @pl.loop(0, EPW // L)
    def _count(i):
        idx = dst_v[pl.ds(i * L, L)]
        plsc.addupdate_scatter(hist_v, [idx], ones)

    pltpu.sync_copy(hist_v, hist_hbm.at[w])


@functools.cache
def _deg_kernel():
    return functools.partial(
        pl.kernel,
        out_type=jax.ShapeDtypeStruct((NC * NS, NPAD), jnp.float32),
        mesh=_mesh(),
        scratch_types=[
            pltpu.VMEM((EPW,), jnp.int32),
            pltpu.VMEM((NPAD,), jnp.float32),
        ],
        compiler_params=pltpu.CompilerParams(needs_layout_passes=False),
    )(_deg_body)


def _make_scatter(edge_split):
    DH = 128
    chunks = CHUNKS // 2 if edge_split else CHUNKS

    def body(table_hbm, src_hbm, dst_hbm, out_hbm,
             src_a, src_b, dst_v, rows_a, rows_b,
             sem_ga, sem_gb, sem_ia, sem_ib, sem_sa, sem_sb, acc):
        c = lax.axis_index("c")
        s = lax.axis_index("s")
        w = c * NS + s if edge_split else s
        pltpu.sync_copy(dst_hbm.at[w], dst_v)

        zeros = jnp.zeros((L,), jnp.float32)

        @pl.loop(0, CHUNK)
        def _zr(r):
            @pl.loop(0, DH // L)
            def _zc(k):
                rows_a[r, pl.ds(k * L, L)] = zeros

        for k in range(RPT // CHUNK):
            pltpu.sync_copy(rows_a, acc.at[pl.ds(s * RPT + k * CHUNK, CHUNK)])

        table = table_hbm if edge_split else table_hbm.at[c]
        last = chunks - 1

        def gather(sbuf, rbuf, sem):
            pltpu.async_copy(table.at[sbuf], rbuf, sem)

        def gwait(rbuf, sem):
            pltpu.make_async_copy(table.at[src_a], rbuf, sem).wait()

        def iload(jj, sbuf, sem):
            pltpu.async_copy(src_hbm.at[w, jj], sbuf, sem)

        def iwait(sbuf, sem):
            pltpu.make_async_copy(src_hbm.at[w, 0], sbuf, sem).wait()

        def scat(rbuf, jj, sem):
            pltpu.async_copy(rbuf, acc.at[dst_v.at[jj]], sem, add=True)

        def swait(rbuf, sem):
            pltpu.make_async_copy(table.at[src_a], rbuf, sem).wait()

        pltpu.sync_copy(src_hbm.at[w, 0], src_a)
        gather(src_a, rows_a, sem_ga)
        iload(1, src_b, sem_ib)
        iwait(src_b, sem_ib)
        plsc.subcore_barrier()

        gwait(rows_a, sem_ga)
        scat(rows_a, 0, sem_sa)
        gather(src_b, rows_b, sem_gb)
        iload(2, src_a, sem_ia)
        gwait(rows_b, sem_gb)
        scat(rows_b, 1, sem_sb)
        swait(rows_a, sem_sa)
        iwait(src_a, sem_ia)
        gather(src_a, rows_a, sem_ga)
        iload(3, src_b, sem_ib)
        iwait(src_b, sem_ib)

        @pl.loop(2, chunks, step=2)
        def _edge(j):
            gwait(rows_a, sem_ga)
            scat(rows_a, j, sem_sa)
            swait(rows_b, sem_sb)
            gather(src_b, rows_b, sem_gb)
            iload(jnp.minimum(j + 2, last), src_a, sem_ia)
            gwait(rows_b, sem_gb)
            scat(rows_b, j + 1, sem_sb)
            swait(rows_a, sem_sa)
            iwait(src_a, sem_ia)
            gather(src_a, rows_a, sem_ga)
            iload(jnp.minimum(j + 3, last), src_b, sem_ib)
            iwait(src_b, sem_ib)

        swait(rows_b, sem_sb)
        gwait(rows_a, sem_ga)
        plsc.subcore_barrier()
        pltpu.sync_copy(acc.at[pl.ds(s * RPT, RPT)],
                        out_hbm.at[c, pl.ds(s * RPT, RPT)])

    tshape = (NPAD, DH) if edge_split else (NC, NPAD, DH)
    nw = NC * NS if edge_split else NS
    return functools.partial(
        pl.kernel,
        out_type=jax.ShapeDtypeStruct((NC, NPAD, DH), jnp.float32),
        mesh=_mesh(),
        scratch_types=[
            pltpu.VMEM((CHUNK,), jnp.int32),
            pltpu.VMEM((CHUNK,), jnp.int32),
            pltpu.VMEM((chunks, CHUNK), jnp.int32),
            pltpu.VMEM((CHUNK, DH), jnp.float32),
            pltpu.VMEM((CHUNK, DH), jnp.float32),
            pltpu.SemaphoreType.DMA,
            pltpu.SemaphoreType.DMA,
            pltpu.SemaphoreType.DMA,
            pltpu.SemaphoreType.DMA,
            pltpu.SemaphoreType.DMA,
            pltpu.SemaphoreType.DMA,
            pltpu.VMEM_SHARED((NPAD, DH), jnp.float32),
        ],
        compiler_params=pltpu.CompilerParams(needs_layout_passes=False),
    )(body)


_scatter_kernel = functools.cache(_make_scatter)


_BM = 512


def _layer1_body(hist_ref, x_ref, w_ref, dinv_ref, ycat_ref):
    i = pl.program_id(0)
    deg = jnp.sum(hist_ref[...], axis=0) + 1.0
    dinv = lax.rsqrt(jnp.maximum(deg, 1.0))
    xw = jnp.dot(x_ref[...], w_ref[...], preferred_element_type=jnp.float32)
    y = xw * dinv[:, None]
    row = i * _BM + lax.broadcasted_iota(jnp.int32, (_BM, 1), 0)
    y = jnp.where(row < N, y, 0.0)
    dinv_ref[...] = dinv
    ycat_ref[0] = y[:, : FEAT // 2]
    ycat_ref[1] = y[:, FEAT // 2:]


def _layer1(hist, x, w):
    return pl.pallas_call(
        _layer1_body,
        grid=(NPAD // _BM,),
        in_specs=[pl.BlockSpec((NC * NS, _BM), lambda i: (0, i)),
                  pl.BlockSpec((_BM, IN_DIM), lambda i: (i, 0)),
                  pl.BlockSpec((IN_DIM, FEAT), lambda i: (0, 0))],
        out_specs=[pl.BlockSpec((_BM,), lambda i: (i,)),
                   pl.BlockSpec((NC, _BM, FEAT // 2), lambda i: (0, i, 0))],
        out_shape=[jax.ShapeDtypeStruct((NPAD,), jnp.float32),
                   jax.ShapeDtypeStruct((NC, NPAD, FEAT // 2), jnp.float32)],
    )(hist, x, w)


def _layer2_body(s1_ref, y1_ref, dinv_ref, b1_ref, wcat_ref, y2_ref, i_ref=None):
    i = pl.program_id(0)
    s = jnp.concatenate([s1_ref[0], s1_ref[1]], axis=1)
    y1 = jnp.concatenate([y1_ref[0], y1_ref[1]], axis=1)
    dinv = dinv_ref[...]
    h = jnp.maximum(dinv[:, None] * (s + y1) + b1_ref[...][None, :], 0.0)
    row = i * _BM + lax.broadcasted_iota(jnp.int32, (_BM, 1), 0)
    h = jnp.where(row < N, h, 0.0)
    c = jnp.dot(h, wcat_ref[...], preferred_element_type=jnp.float32)
    y2_ref[...] = c * dinv[:, None]


def _layer2(s1cat, y1cat, dinv, b1, wcat):
    return pl.pallas_call(
        _layer2_body,
        grid=(NPAD // _BM,),
        in_specs=[pl.BlockSpec((NC, _BM, FEAT // 2), lambda i: (0, i, 0)),
                  pl.BlockSpec((NC, _BM, FEAT // 2), lambda i: (0, i, 0)),
                  pl.BlockSpec((_BM,), lambda i: (i,)),
                  pl.BlockSpec((FEAT,), lambda i: (0,)),
                  pl.BlockSpec((FEAT, 2 * LAT), lambda i: (0, 0))],
        out_specs=pl.BlockSpec((_BM, 2 * LAT), lambda i: (i, 0)),
        out_shape=jax.ShapeDtypeStruct((NPAD, 2 * LAT), jnp.float32),
    )(s1cat, y1cat, dinv, b1, wcat)


def _zcomp_body(s2_ref, y2_ref, dinv_ref, bcat_ref, gn_ref, z_ref):
    s = s2_ref[0] + s2_ref[1]
    y2 = y2_ref[...]
    o = dinv_ref[...][:, None] * (s + y2) + bcat_ref[...][None, :]
    xu = o[:, :LAT]
    xs = o[:, LAT:]
    z = gn_ref[...] * jnp.exp(xs) + xu
    z_ref[...] = (z * 0.7071067811865476).astype(jnp.bfloat16)


def _zcomp(s2cat, y2cat, dinv, bcat, gn):
    return pl.pallas_call(
        _zcomp_body,
        grid=(NPAD // _BM,),
        in_specs=[pl.BlockSpec((NC, _BM, 2 * LAT), lambda i: (0, i, 0)),
                  pl.BlockSpec((_BM, 2 * LAT), lambda i: (i, 0)),
                  pl.BlockSpec((_BM,), lambda i: (i,)),
                  pl.BlockSpec((2 * LAT,), lambda i: (0,)),
                  pl.BlockSpec((_BM, LAT), lambda i: (i, 0))],
        out_specs=pl.BlockSpec((_BM, LAT), lambda i: (i, 0)),
        out_shape=jax.ShapeDtypeStruct((NPAD, LAT), jnp.bfloat16),
    )(s2cat, y2cat, dinv, bcat, gn)


_BD = 2048
_BDN = 2048


def _decode_body(zr_ref, zc_ref, o_ref):
    p = lax.dot_general(zr_ref[...], zc_ref[...],
                        (((1,), (1,)), ((), ())),
                        preferred_element_type=jnp.float32)
    o_ref[...] = 0.5 * jnp.tanh(p) + 0.5


def _decode(z):
    return pl.pallas_call(
        _decode_body,
        grid=(NPAD // _BD, NPAD // _BDN),
        in_specs=[pl.BlockSpec((_BD, LAT), lambda i, j: (i, 0)),
                  pl.BlockSpec((_BDN, LAT), lambda i, j: (j, 0))],
        out_specs=pl.BlockSpec((_BD, _BDN), lambda i, j: (i, j)),
        out_shape=jax.ShapeDtypeStruct((N, N), jnp.float32),
    )(z, z)


def kernel(x, edge_index, W1, b1, Wmu, bmu, Wsig, bsig, gnoise):
    ei = edge_index.astype(jnp.int32)
    pad = N + jnp.arange(EPAD - E, dtype=jnp.int32) % (NPAD - N)
    src = jnp.concatenate([ei[0], pad])
    dst = jnp.concatenate([ei[1], pad])
    src_sc = src.reshape(NS, CHUNKS, CHUNK)
    dst_sc = dst.reshape(NS, CHUNKS, CHUNK)
    src_es = src.reshape(NC * NS, CHUNKS // 2, CHUNK)
    dst_es = dst.reshape(NC * NS, CHUNKS // 2, CHUNK)
    dst_deg = dst.reshape(NC * NS, EPW)

    wcat = jnp.concatenate([Wmu, Wsig], axis=1)
    bcat = jnp.concatenate([bmu, bsig])

    hist = _deg_kernel()(dst_deg)
    dinv, y1cat = _layer1(hist, x, W1)
    s1cat = _scatter_kernel(False)(y1cat, src_sc, dst_sc)
    y2 = _layer2(s1cat, y1cat, dinv, b1, wcat)
    s2cat = _scatter_kernel(True)(y2, src_es, dst_es)
    z = _zcomp(s2cat, y2, dinv, bcat, gnoise)
    return _decode(z)


# speedup vs baseline: 16.0022x; 1.0020x over previous
import functools

import jax
import jax.numpy as jnp
from jax import lax
from jax.experimental import pallas as pl
from jax.experimental.pallas import tpu as pltpu
from jax.experimental.pallas import tpu_sc as plsc

N = 10000
NPAD = 10240
IN_DIM = 256
FEAT = 256
LAT = 64
E = 160000
EPAD = 163840

NC, NS, L = 2, 16, 16
CHUNK = 128
CHUNKS = EPAD // NS // CHUNK
RPT = NPAD // NS
EPW = EPAD // (NC * NS)


def _mesh():
    return plsc.VectorSubcoreMesh(
        core_axis_name="c", subcore_axis_name="s",
        num_cores=NC, num_subcores=NS)


def _deg_body(dst_hbm, hist_hbm, dst_v, hist_v):
    c = lax.axis_index("c")
    s = lax.axis_index("s")
    w = s * NC + c
    pltpu.sync_copy(dst_hbm.at[w], dst_v)
    zeros = jnp.zeros((L,), jnp.float32)
    ones = jnp.ones((L,), jnp.float32)

    @pl.loop(0, NPAD // L)
    def _zero(i):
        hist_v[pl.ds(i * L, L)] = zeros

    @pl.loop(0, EPW // L)
    def _count(i):
        idx = dst_v[pl.ds(i * L, L)]
        plsc.addupdate_scatter(hist_v, [idx], ones)

    pltpu.sync_copy(hist_v, hist_hbm.at[w])


@functools.cache
def _deg_kernel():
    return functools.partial(
        pl.kernel,
        out_type=jax.ShapeDtypeStruct((NC * NS, NPAD), jnp.float32),
        mesh=_mesh(),
        scratch_types=[
            pltpu.VMEM((EPW,), jnp.int32),
            pltpu.VMEM((NPAD,), jnp.float32),
        ],
        compiler_params=pltpu.CompilerParams(needs_layout_passes=False),
    )(_deg_body)


def _make_scatter(edge_split):
    DH = 128
    chunks = CHUNKS // 2 if edge_split else CHUNKS

    def body(table_hbm, src_hbm, dst_hbm, out_hbm,
             src_a, src_b, dst_v, rows_a, rows_b,
             sem_ga, sem_gb, sem_ia, sem_ib, sem_sa, sem_sb, acc):
        c = lax.axis_index("c")
        s = lax.axis_index("s")
        w = c * NS + s if edge_split else s
        pltpu.sync_copy(dst_hbm.at[w], dst_v)

        zeros = jnp.zeros((L,), jnp.float32)

        @pl.loop(0, CHUNK)
        def _zr(r):
            @pl.loop(0, DH // L)
            def _zc(k):
                rows_a[r, pl.ds(k * L, L)] = zeros

        for k in range(RPT // CHUNK):
            pltpu.sync_copy(rows_a, acc.at[pl.ds(s * RPT + k * CHUNK, CHUNK)])

        table = table_hbm if edge_split else table_hbm.at[c]
        last = chunks - 1

        def gather(sbuf, rbuf, sem):
            pltpu.async_copy(table.at[sbuf], rbuf, sem)

        def gwait(rbuf, sem):
            pltpu.make_async_copy(table.at[src_a], rbuf, sem).wait()

        def iload(jj, sbuf, sem):
            pltpu.async_copy(src_hbm.at[w, jj], sbuf, sem)

        def iwait(sbuf, sem):
            pltpu.make_async_copy(src_hbm.at[w, 0], sbuf, sem).wait()

        def scat(rbuf, jj, sem):
            pltpu.async_copy(rbuf, acc.at[dst_v.at[jj]], sem, add=True)

        def swait(rbuf, sem):
            pltpu.make_async_copy(table.at[src_a], rbuf, sem).wait()

        pltpu.sync_copy(src_hbm.at[w, 0], src_a)
        gather(src_a, rows_a, sem_ga)
        iload(1, src_b, sem_ib)
        iwait(src_b, sem_ib)
        plsc.subcore_barrier()

        gwait(rows_a, sem_ga)
        scat(rows_a, 0, sem_sa)
        gather(src_b, rows_b, sem_gb)
        iload(2, src_a, sem_ia)
        gwait(rows_b, sem_gb)
        scat(rows_b, 1, sem_sb)
        swait(rows_a, sem_sa)
        iwait(src_a, sem_ia)
        gather(src_a, rows_a, sem_ga)
        iload(3, src_b, sem_ib)
        iwait(src_b, sem_ib)

        @pl.loop(2, chunks, step=2)
        def _edge(j):
            gwait(rows_a, sem_ga)
            scat(rows_a, j, sem_sa)
            swait(rows_b, sem_sb)
            gather(src_b, rows_b, sem_gb)
            iload(jnp.minimum(j + 2, last), src_a, sem_ia)
            gwait(rows_b, sem_gb)
            scat(rows_b, j + 1, sem_sb)
            swait(rows_a, sem_sa)
            iwait(src_a, sem_ia)
            gather(src_a, rows_a, sem_ga)
            iload(jnp.minimum(j + 3, last), src_b, sem_ib)
            iwait(src_b, sem_ib)

        swait(rows_b, sem_sb)
        gwait(rows_a, sem_ga)
        plsc.subcore_barrier()
        pltpu.sync_copy(acc.at[pl.ds(s * RPT, RPT)],
                        out_hbm.at[c, pl.ds(s * RPT, RPT)])

    tshape = (NPAD, DH) if edge_split else (NC, NPAD, DH)
    nw = NC * NS if edge_split else NS
    return functools.partial(
        pl.kernel,
        out_type=jax.ShapeDtypeStruct((NC, NPAD, DH), jnp.float32),
        mesh=_mesh(),
        scratch_types=[
            pltpu.VMEM((CHUNK,), jnp.int32),
            pltpu.VMEM((CHUNK,), jnp.int32),
            pltpu.VMEM((chunks, CHUNK), jnp.int32),
            pltpu.VMEM((CHUNK, DH), jnp.float32),
            pltpu.VMEM((CHUNK, DH), jnp.float32),
            pltpu.SemaphoreType.DMA,
            pltpu.SemaphoreType.DMA,
            pltpu.SemaphoreType.DMA,
            pltpu.SemaphoreType.DMA,
            pltpu.SemaphoreType.DMA,
            pltpu.SemaphoreType.DMA,
            pltpu.VMEM_SHARED((NPAD, DH), jnp.float32),
        ],
        compiler_params=pltpu.CompilerParams(needs_layout_passes=False),
    )(body)


_scatter_kernel = functools.cache(_make_scatter)


_BM = 512


def _prescale_body(hist_ref, x_ref, dinv_ref, ycat_ref):
    i = pl.program_id(0)
    deg = jnp.sum(hist_ref[...], axis=0) + 1.0
    dinv = lax.rsqrt(jnp.maximum(deg, 1.0))
    y = x_ref[...] * dinv[:, None]
    row = i * _BM + lax.broadcasted_iota(jnp.int32, (_BM, 1), 0)
    y = jnp.where(row < N, y, 0.0)
    dinv_ref[...] = dinv
    ycat_ref[0] = y[:, : IN_DIM // 2]
    ycat_ref[1] = y[:, IN_DIM // 2:]


def _prescale(hist, x):
    return pl.pallas_call(
        _prescale_body,
        grid=(NPAD // _BM,),
        in_specs=[pl.BlockSpec((NC * NS, _BM), lambda i: (0, i)),
                  pl.BlockSpec((_BM, IN_DIM), lambda i: (i, 0))],
        out_specs=[pl.BlockSpec((_BM,), lambda i: (i,)),
                   pl.BlockSpec((NC, _BM, IN_DIM // 2), lambda i: (0, i, 0))],
        out_shape=[jax.ShapeDtypeStruct((NPAD,), jnp.float32),
                   jax.ShapeDtypeStruct((NC, NPAD, IN_DIM // 2), jnp.float32)],
    )(hist, x)


def _enc_body(sx_ref, yx_ref, dinv_ref, b1_ref, w1_ref, wcat_ref, y2_ref):
    i = pl.program_id(0)
    sx = jnp.concatenate([sx_ref[0], sx_ref[1]], axis=1)
    yx = jnp.concatenate([yx_ref[0], yx_ref[1]], axis=1)
    dinv = dinv_ref[...]
    pre = dinv[:, None] * (sx + yx)
    h = jnp.maximum(
        jnp.dot(pre, w1_ref[...], preferred_element_type=jnp.float32)
        + b1_ref[...][None, :], 0.0)
    row = i * _BM + lax.broadcasted_iota(jnp.int32, (_BM, 1), 0)
    h = jnp.where(row < N, h, 0.0)
    c = jnp.dot(h, wcat_ref[...], preferred_element_type=jnp.float32)
    y2_ref[...] = c * dinv[:, None]


def _enc(sxcat, yxcat, dinv, b1, w1, wcat):
    return pl.pallas_call(
        _enc_body,
        grid=(NPAD // _BM,),
        in_specs=[pl.BlockSpec((NC, _BM, IN_DIM // 2), lambda i: (0, i, 0)),
                  pl.BlockSpec((NC, _BM, IN_DIM // 2), lambda i: (0, i, 0)),
                  pl.BlockSpec((_BM,), lambda i: (i,)),
                  pl.BlockSpec((FEAT,), lambda i: (0,)),
                  pl.BlockSpec((IN_DIM, FEAT), lambda i: (0, 0)),
                  pl.BlockSpec((FEAT, 2 * LAT), lambda i: (0, 0))],
        out_specs=pl.BlockSpec((_BM, 2 * LAT), lambda i: (i, 0)),
        out_shape=jax.ShapeDtypeStruct((NPAD, 2 * LAT), jnp.float32),
    )(sxcat, yxcat, dinv, b1, w1, wcat)


def _zcomp_body(s2_ref, y2_ref, dinv_ref, bcat_ref, gn_ref, z_ref):
    s = s2_ref[0] + s2_ref[1]
    y2 = y2_ref[...]
    o = dinv_ref[...][:, None] * (s + y2) + bcat_ref[...][None, :]
    xu = o[:, :LAT]
    xs = o[:, LAT:]
    z = gn_ref[...] * jnp.exp(xs) + xu
    z_ref[...] = (z * 0.7071067811865476).astype(jnp.bfloat16)


def _zcomp(s2cat, y2cat, dinv, bcat, gn):
    return pl.pallas_call(
        _zcomp_body,
        grid=(NPAD // _BM,),
        in_specs=[pl.BlockSpec((NC, _BM, 2 * LAT), lambda i: (0, i, 0)),
                  pl.BlockSpec((_BM, 2 * LAT), lambda i: (i, 0)),
                  pl.BlockSpec((_BM,), lambda i: (i,)),
                  pl.BlockSpec((2 * LAT,), lambda i: (0,)),
                  pl.BlockSpec((_BM, LAT), lambda i: (i, 0))],
        out_specs=pl.BlockSpec((_BM, LAT), lambda i: (i, 0)),
        out_shape=jax.ShapeDtypeStruct((NPAD, LAT), jnp.bfloat16),
    )(s2cat, y2cat, dinv, bcat, gn)


_BD = 2048
_BDN = 2048


def _decode_body(zr_ref, zc_ref, o_ref):
    p = lax.dot_general(zr_ref[...], zc_ref[...],
                        (((1,), (1,)), ((), ())),
                        preferred_element_type=jnp.float32)
    o_ref[...] = 0.5 * jnp.tanh(p) + 0.5


def _decode(z):
    return pl.pallas_call(
        _decode_body,
        grid=(NPAD // _BD, NPAD // _BDN),
        in_specs=[pl.BlockSpec((_BD, LAT), lambda i, j: (i, 0)),
                  pl.BlockSpec((_BDN, LAT), lambda i, j: (j, 0))],
        out_specs=pl.BlockSpec((_BD, _BDN), lambda i, j: (i, j)),
        out_shape=jax.ShapeDtypeStruct((N, N), jnp.float32),
    )(z, z)


def kernel(x, edge_index, W1, b1, Wmu, bmu, Wsig, bsig, gnoise):
    ei = edge_index.astype(jnp.int32)
    pad = N + jnp.arange(EPAD - E, dtype=jnp.int32) % (NPAD - N)
    src = jnp.concatenate([ei[0], pad])
    dst = jnp.concatenate([ei[1], pad])
    src_sc = src.reshape(NS, CHUNKS, CHUNK)
    dst_sc = dst.reshape(NS, CHUNKS, CHUNK)
    src_es = src.reshape(NC * NS, CHUNKS // 2, CHUNK)
    dst_es = dst.reshape(NC * NS, CHUNKS // 2, CHUNK)
    dst_deg = dst.reshape(NC * NS, EPW)

    wcat = jnp.concatenate([Wmu, Wsig], axis=1)
    bcat = jnp.concatenate([bmu, bsig])

    hist = _deg_kernel()(dst_deg)
    dinv, yxcat = _prescale(hist, x)
    sxcat = _scatter_kernel(False)(yxcat, src_sc, dst_sc)
    y2 = _enc(sxcat, yxcat, dinv, b1, W1, wcat)
    s2cat = _scatter_kernel(True)(y2, src_es, dst_es)
    z = _zcomp(s2cat, y2, dinv, bcat, gnoise)
    return _decode(z)
